# baseline (speedup 1.0000x reference)
import functools

import jax
import jax.numpy as jnp
from jax import lax
from jax.experimental import pallas as pl
from jax.experimental.pallas import tpu as pltpu
from jax.experimental.pallas import tpu_sc as plsc

V_DIM = 1000000
D = 64
B = 16384
L = 20
LANES = 16
NC, NS = 2, 16
NW = NC * NS
BPW = B // NW
BC = 16
NCHUNK = BPW // BC
RPC = BC * L
GROUP = 80
NGROUP = RPC // GROUP
CROWS = BC * D
CGROUP = 128
NCGROUP = CROWS // CGROUP
ROW16 = V_DIM // LANES


def _sc_dots(center, pos_flat, neg_flat, wcen16, w_context):
    mesh = plsc.VectorSubcoreMesh(
        core_axis_name="c", subcore_axis_name="s",
        num_cores=NC, num_subcores=NS)
    out_t = (jax.ShapeDtypeStruct((B * L,), jnp.float32),
             jax.ShapeDtypeStruct((B * L,), jnp.float32))
    scratch = [
        pltpu.VMEM((BC,), jnp.int32),
        pltpu.VMEM((LANES,), jnp.int32),
        pltpu.VMEM((CROWS,), jnp.int32),
        pltpu.VMEM((RPC,), jnp.int32),
        pltpu.VMEM((RPC,), jnp.int32),
        pltpu.VMEM((CROWS, LANES), jnp.float32),
        pltpu.VMEM((RPC, D), jnp.float32),
        pltpu.VMEM((RPC, D), jnp.float32),
        pltpu.VMEM((RPC,), jnp.float32),
        pltpu.VMEM((RPC,), jnp.float32),
        pltpu.VMEM((LANES * LANES,), jnp.float32),
        pltpu.SemaphoreType.DMA,
        pltpu.SemaphoreType.DMA,
        pltpu.SemaphoreType.DMA,
    ]

    @functools.partial(pl.kernel, out_type=out_t, mesh=mesh,
                       scratch_types=scratch,
                       compiler_params=pltpu.CompilerParams(
                           use_tc_tiling_on_sc=False,
                           needs_layout_passes=False))
    def k(center_h, pos_h, neg_h, wcen_h, wctx_h, pdots_h, ndots_h,
          cidx_v, clane_v, cgidx_v, pidx_v, nidx_v, crows_v, p_v, n_v,
          ps_v, ns_v, tmp_v, sem_a, sem_b, sem_c):
        wid = lax.axis_index("s") * NC + lax.axis_index("c")
        iota = lax.iota(jnp.int32, LANES)
        scat_base = iota * LANES
        iota16 = iota * LANES

        def table(rows_v, idx_v, out_v):
            def b_body(b, carry):
                lane = plsc.load_gather(clane_v, [jnp.full((LANES,), b,
                                                           jnp.int32)])
                cs = []
                for kk in range(4):
                    jvec = iota16 + (kk * 256 + b)
                    cs.append(plsc.load_gather(crows_v, [jvec, lane]))
                for l in range(L):
                    r = b * L + l
                    a = (rows_v[r, pl.ds(0, LANES)] * cs[0]
                         + rows_v[r, pl.ds(LANES, LANES)] * cs[1]
                         + rows_v[r, pl.ds(2 * LANES, LANES)] * cs[2]
                         + rows_v[r, pl.ds(3 * LANES, LANES)] * cs[3])
                    col = lax.rem(r, LANES)
                    plsc.store_scatter(tmp_v, [scat_base + col], a)

                    @pl.when(col == LANES - 1)
                    def _flush():
                        w0 = r - (LANES - 1)
                        s = tmp_v[pl.ds(0, LANES)]
                        for kk in range(1, LANES):
                            s = s + tmp_v[pl.ds(kk * LANES, LANES)]
                        rvec = w0 + iota
                        bvec = rvec // L
                        m = ((idx_v[pl.ds(w0, LANES)] != 0)
                             & (plsc.load_gather(cidx_v, [bvec]) != 0))
                        out_v[pl.ds(w0, LANES)] = jnp.where(m, s, 0.0)

                return carry

            lax.fori_loop(0, BC, b_body, 0)

        def chunk_body(t, carry):
            b0 = wid * BPW + t * BC
            r0 = b0 * L
            pltpu.sync_copy(center_h.at[pl.ds(b0, BC)], cidx_v)
            pltpu.sync_copy(pos_h.at[pl.ds(r0, RPC)], pidx_v)
            pltpu.sync_copy(neg_h.at[pl.ds(r0, RPC)], nidx_v)
            cvec = cidx_v[pl.ds(0, LANES)]
            clane_v[pl.ds(0, LANES)] = lax.rem(cvec, LANES)
            base = lax.div(cvec, LANES)

            def cg_body(dd, carry2):
                cgidx_v[pl.ds(dd * LANES, LANES)] = base + dd * ROW16
                return carry2

            lax.fori_loop(0, D, cg_body, 0)
            ccs = [pltpu.async_copy(
                wcen_h.at[cgidx_v.at[pl.ds(g * CGROUP, CGROUP)]],
                crows_v.at[pl.ds(g * CGROUP, CGROUP), :], sem_c)
                for g in range(NCGROUP)]
            pcs = [pltpu.async_copy(
                wctx_h.at[pidx_v.at[pl.ds(g * GROUP, GROUP)]],
                p_v.at[pl.ds(g * GROUP, GROUP), :], sem_a)
                for g in range(NGROUP)]
            ncs = [pltpu.async_copy(
                wctx_h.at[nidx_v.at[pl.ds(g * GROUP, GROUP)]],
                n_v.at[pl.ds(g * GROUP, GROUP), :], sem_b)
                for g in range(NGROUP)]
            for cp in ccs:
                cp.wait()
            for cp in pcs:
                cp.wait()
            table(p_v, pidx_v, ps_v)
            for cp in ncs:
                cp.wait()
            table(n_v, nidx_v, ns_v)
            pltpu.sync_copy(ps_v, pdots_h.at[pl.ds(r0, RPC)])
            pltpu.sync_copy(ns_v, ndots_h.at[pl.ds(r0, RPC)])
            return carry

        lax.fori_loop(0, NCHUNK, chunk_body, 0)

    return k(center, pos_flat, neg_flat, wcen16, w_context)


_ROWS, _COLS = 640, 512
_BLK = 64


def _tc_loss(pdots, ndots):
    def body(p_ref, n_ref, o_ref):
        i = pl.program_id(0)

        @pl.when(i == 0)
        def _init():
            o_ref[0, 0] = 0.0

        def ls(x):
            return jnp.minimum(x, 0.0) - jnp.log(1.0 + jnp.exp(-jnp.abs(x)))

        o_ref[0, 0] += jnp.sum(ls(p_ref[...])) + jnp.sum(ls(-n_ref[...]))

        @pl.when(i == pl.num_programs(0) - 1)
        def _fin():
            o_ref[0, 0] = o_ref[0, 0] * (-1.0 / B)

    out = pl.pallas_call(
        body,
        grid=(_ROWS // _BLK,),
        in_specs=[pl.BlockSpec((_BLK, _COLS), lambda i: (i, 0)),
                  pl.BlockSpec((_BLK, _COLS), lambda i: (i, 0))],
        out_specs=pl.BlockSpec(memory_space=pltpu.SMEM),
        out_shape=jax.ShapeDtypeStruct((1, 1), jnp.float32),
    )(pdots.reshape(_ROWS, _COLS), ndots.reshape(_ROWS, _COLS))
    return out[0, 0]


def kernel(center, pos, neg, W_center, W_context):
    wcen16 = W_center.T.reshape(V_DIM * D // LANES, LANES)
    pdots, ndots = _sc_dots(center, pos.reshape(-1), neg.reshape(-1),
                            wcen16, W_context)
    return _tc_loss(pdots, ndots)


# speedup vs baseline: 4.0388x; 4.0388x over previous
import functools

import jax
import jax.numpy as jnp
from jax import lax
from jax.experimental import pallas as pl
from jax.experimental.pallas import tpu as pltpu
from jax.experimental.pallas import tpu_sc as plsc

V_DIM = 1000000
D = 64
B = 16384
L = 20
LANES = 16
NC, NS = 2, 16
NW = NC * NS
BPW = B // NW
BC = 32
NCHUNK = BPW // BC
RPC = BC * L


def _sc_dots(center, pos_t, neg_t, w_center, w_context):
    mesh = plsc.VectorSubcoreMesh(
        core_axis_name="c", subcore_axis_name="s",
        num_cores=NC, num_subcores=NS)
    out_t = (jax.ShapeDtypeStruct((B * L,), jnp.float32),
             jax.ShapeDtypeStruct((B * L,), jnp.float32))
    scratch = [
        pltpu.VMEM((BC,), jnp.int32),
        pltpu.VMEM((RPC,), jnp.int32),
        pltpu.VMEM((RPC,), jnp.int32),
        pltpu.VMEM((BC, D), jnp.float32),
        pltpu.VMEM((RPC, D), jnp.float32),
        pltpu.VMEM((RPC, D), jnp.float32),
        pltpu.VMEM((RPC,), jnp.float32),
        pltpu.VMEM((RPC,), jnp.float32),
        pltpu.VMEM((LANES * LANES,), jnp.float32),
        pltpu.SemaphoreType.DMA,
        pltpu.SemaphoreType.DMA,
        pltpu.SemaphoreType.DMA,
    ]

    @functools.partial(pl.kernel, out_type=out_t, mesh=mesh,
                       scratch_types=scratch,
                       compiler_params=pltpu.CompilerParams(
                           use_tc_tiling_on_sc=False,
                           needs_layout_passes=False))
    def k(center_h, pos_h, neg_h, wcen_h, wctx_h, pdots_h, ndots_h,
          cidx_v, pidx_v, nidx_v, c_v, p_v, n_v, ps_v, ns_v, tmp_v,
          sem_a, sem_b, sem_i):
        wid = lax.axis_index("s") * NC + lax.axis_index("c")
        iota = lax.iota(jnp.int32, LANES)
        scat_base = iota * LANES

        def table(rows_v, idx_v, out_v):
            def b_body(b, carry):
                c0 = c_v[b, pl.ds(0, LANES)]
                c1 = c_v[b, pl.ds(LANES, LANES)]
                c2 = c_v[b, pl.ds(2 * LANES, LANES)]
                c3 = c_v[b, pl.ds(3 * LANES, LANES)]
                for l in range(L):
                    r = b * L + l
                    j = l * BC + b
                    a = (rows_v[j, pl.ds(0, LANES)] * c0
                         + rows_v[j, pl.ds(LANES, LANES)] * c1
                         + rows_v[j, pl.ds(2 * LANES, LANES)] * c2
                         + rows_v[j, pl.ds(3 * LANES, LANES)] * c3)
                    col = lax.rem(r, LANES)
                    plsc.store_scatter(tmp_v, [scat_base + col], a)

                    @pl.when(col == LANES - 1)
                    def _flush():
                        w0 = r - (LANES - 1)
                        s = tmp_v[pl.ds(0, LANES)]
                        for kk in range(1, LANES):
                            s = s + tmp_v[pl.ds(kk * LANES, LANES)]
                        rvec = w0 + iota
                        bvec = rvec // L
                        lvec = rvec - bvec * L
                        m = ((plsc.load_gather(
                                  idx_v, [lvec * BC + bvec]) != 0)
                             & (plsc.load_gather(cidx_v, [bvec]) != 0))
                        out_v[pl.ds(w0, LANES)] = jnp.where(m, s, 0.0)

                return carry

            lax.fori_loop(0, BC, b_body, 0)

        def chunk_body(t, carry):
            b0 = wid * BPW + t * BC
            r0 = b0 * L
            ics = [pltpu.async_copy(
                pos_h.at[l, pl.ds(b0, BC)],
                pidx_v.at[pl.ds(l * BC, BC)], sem_i)
                for l in range(L)]
            ics += [pltpu.async_copy(
                neg_h.at[l, pl.ds(b0, BC)],
                nidx_v.at[pl.ds(l * BC, BC)], sem_i)
                for l in range(L)]
            pltpu.sync_copy(center_h.at[pl.ds(b0, BC)], cidx_v)
            for c in ics:
                c.wait()
            cp = pltpu.async_copy(wcen_h.at[cidx_v], c_v, sem_a)
            pcs = [pltpu.async_copy(
                wctx_h.at[pidx_v.at[pl.ds(g * 128, 128)]],
                p_v.at[pl.ds(g * 128, 128), :], sem_a)
                for g in range(RPC // 128)]
            ncs = [pltpu.async_copy(
                wctx_h.at[nidx_v.at[pl.ds(g * 128, 128)]],
                n_v.at[pl.ds(g * 128, 128), :], sem_b)
                for g in range(RPC // 128)]
            cp.wait()
            for c in pcs:
                c.wait()
            table(p_v, pidx_v, ps_v)
            for c in ncs:
                c.wait()
            table(n_v, nidx_v, ns_v)
            pltpu.sync_copy(ps_v, pdots_h.at[pl.ds(r0, RPC)])
            pltpu.sync_copy(ns_v, ndots_h.at[pl.ds(r0, RPC)])
            return carry

        lax.fori_loop(0, NCHUNK, chunk_body, 0)

    return k(center, pos_t, neg_t, w_center, w_context)


_ROWS, _COLS = 640, 512
_BLK = 64


def _tc_loss(pdots, ndots):
    def body(p_ref, n_ref, o_ref):
        i = pl.program_id(0)

        @pl.when(i == 0)
        def _init():
            o_ref[0, 0] = 0.0

        def ls(x):
            return jnp.minimum(x, 0.0) - jnp.log(1.0 + jnp.exp(-jnp.abs(x)))

        o_ref[0, 0] += jnp.sum(ls(p_ref[...])) + jnp.sum(ls(-n_ref[...]))

        @pl.when(i == pl.num_programs(0) - 1)
        def _fin():
            o_ref[0, 0] = o_ref[0, 0] * (-1.0 / B)

    out = pl.pallas_call(
        body,
        grid=(_ROWS // _BLK,),
        in_specs=[pl.BlockSpec((_BLK, _COLS), lambda i: (i, 0)),
                  pl.BlockSpec((_BLK, _COLS), lambda i: (i, 0))],
        out_specs=pl.BlockSpec(memory_space=pltpu.SMEM),
        out_shape=jax.ShapeDtypeStruct((1, 1), jnp.float32),
    )(pdots.reshape(_ROWS, _COLS), ndots.reshape(_ROWS, _COLS))
    return out[0, 0]


def kernel(center, pos, neg, W_center, W_context):
    pdots, ndots = _sc_dots(center, pos.T, neg.T, W_center, W_context)
    return _tc_loss(pdots, ndots)


# speedup vs baseline: 4.2043x; 1.0410x over previous
import functools

import jax
import jax.numpy as jnp
from jax import lax
from jax.experimental import pallas as pl
from jax.experimental.pallas import tpu as pltpu
from jax.experimental.pallas import tpu_sc as plsc

V_DIM = 1000000
D = 64
B = 16384
L = 20
LANES = 16
NC, NS = 2, 16
NW = NC * NS
BPW = B // NW
BC = 16
NCHUNK = BPW // BC
RPC = BC * L
NG = RPC // 128
_SC_PARAMS = pltpu.CompilerParams(use_tc_tiling_on_sc=False,
                                  needs_layout_passes=False)

GRP = 80
NGRP = RPC // GRP


def _sc_center(center, w_center):
    mesh = plsc.VectorSubcoreMesh(
        core_axis_name="c", subcore_axis_name="s",
        num_cores=NC, num_subcores=NS)

    @functools.partial(
        pl.kernel, mesh=mesh,
        out_type=jax.ShapeDtypeStruct((B, D), jnp.float32),
        scratch_types=[
            pltpu.VMEM((BPW,), jnp.int32),
            pltpu.VMEM((BPW, D), jnp.float32),
            pltpu.SemaphoreType.DMA,
        ],
        compiler_params=_SC_PARAMS)
    def kc(center_h, wcen_h, call_h, cidx_v, cbuf_v, sem):
        wid = lax.axis_index("s") * NC + lax.axis_index("c")
        b0 = wid * BPW
        pltpu.sync_copy(center_h.at[pl.ds(b0, BPW)], cidx_v)
        cps = [pltpu.async_copy(
            wcen_h.at[cidx_v.at[pl.ds(g * 128, 128)]],
            cbuf_v.at[pl.ds(g * 128, 128), :], sem)
            for g in range(BPW // 128)]
        for cp in cps:
            cp.wait()
        pltpu.sync_copy(cbuf_v, call_h.at[pl.ds(b0, BPW), :])

    return kc(center, w_center)


def _sc_dots(c_all, pos_t, neg_t, center, w_context):
    mesh = plsc.VectorSubcoreMesh(
        core_axis_name="c", subcore_axis_name="s",
        num_cores=NC, num_subcores=NS)
    out_t = (jax.ShapeDtypeStruct((B * L,), jnp.float32),
             jax.ShapeDtypeStruct((B * L,), jnp.float32))
    scratch = [
        pltpu.VMEM((2, BC), jnp.int32),
        pltpu.VMEM((2, RPC), jnp.int32),
        pltpu.VMEM((2, RPC), jnp.int32),
        pltpu.VMEM((2, BC, D), jnp.float32),
        pltpu.VMEM((2, RPC, D), jnp.float32),
        pltpu.VMEM((2, RPC, D), jnp.float32),
        pltpu.VMEM((RPC,), jnp.float32),
        pltpu.VMEM((RPC,), jnp.float32),
        pltpu.VMEM((LANES * LANES,), jnp.float32),
        pltpu.SemaphoreType.DMA,
        pltpu.SemaphoreType.DMA,
        pltpu.SemaphoreType.DMA,
        pltpu.SemaphoreType.DMA,
        pltpu.SemaphoreType.DMA,
        pltpu.SemaphoreType.DMA,
    ]

    @functools.partial(pl.kernel, out_type=out_t, mesh=mesh,
                       scratch_types=scratch,
                       compiler_params=_SC_PARAMS)
    def k(call_h, pos_h, neg_h, center_h, wctx_h, pdots_h, ndots_h,
          cidx_v, pidx_v, nidx_v, c_v, p_v, n_v, ps_v, ns_v, tmp_v,
          semp0, semp1, semn0, semn1, semi0, semi1):
        wid = lax.axis_index("s") * NC + lax.axis_index("c")
        iota = lax.iota(jnp.int32, LANES)
        scat_base = iota * LANES
        semp = [semp0, semp1]
        semn = [semn0, semn1]
        semi = [semi0, semi1]

        def stage_idx(t, u):
            b0 = wid * BPW + t * BC
            pltpu.async_copy(center_h.at[pl.ds(b0, BC)],
                             cidx_v.at[u], semi[u])
            for l in range(L):
                pltpu.async_copy(pos_h.at[l, pl.ds(b0, BC)],
                                 pidx_v.at[u, pl.ds(l * BC, BC)], semi[u])
                pltpu.async_copy(neg_h.at[l, pl.ds(b0, BC)],
                                 nidx_v.at[u, pl.ds(l * BC, BC)], semi[u])

        def wait_idx(u):
            pltpu.make_async_copy(center_h.at[pl.ds(0, BC)],
                                  cidx_v.at[u], semi[u]).wait()
            for l in range(L):
                pltpu.make_async_copy(
                    pos_h.at[0, pl.ds(0, BC)],
                    pidx_v.at[u, pl.ds(l * BC, BC)], semi[u]).wait()
                pltpu.make_async_copy(
                    neg_h.at[0, pl.ds(0, BC)],
                    nidx_v.at[u, pl.ds(l * BC, BC)], semi[u]).wait()

        def fire_gathers(t, u):
            b0 = wid * BPW + t * BC
            pltpu.async_copy(call_h.at[pl.ds(b0, BC), :],
                             c_v.at[u], semp[u])
            for g in range(NGRP):
                pltpu.async_copy(
                    wctx_h.at[pidx_v.at[u, pl.ds(g * GRP, GRP)]],
                    p_v.at[u, pl.ds(g * GRP, GRP), :], semp[u])
                pltpu.async_copy(
                    wctx_h.at[nidx_v.at[u, pl.ds(g * GRP, GRP)]],
                    n_v.at[u, pl.ds(g * GRP, GRP), :], semn[u])

        def wait_pos(u):
            pltpu.make_async_copy(call_h.at[pl.ds(0, BC), :],
                                  c_v.at[u], semp[u]).wait()
            for g in range(NGRP):
                pltpu.make_async_copy(
                    wctx_h.at[pl.ds(0, GRP), :],
                    p_v.at[u, pl.ds(g * GRP, GRP), :], semp[u]).wait()

        def wait_neg(u):
            for g in range(NGRP):
                pltpu.make_async_copy(
                    wctx_h.at[pl.ds(0, GRP), :],
                    n_v.at[u, pl.ds(g * GRP, GRP), :], semn[u]).wait()

        def table(u, rows_v, idx_v, out_v):
            def b_body(b, carry):
                c0 = c_v[u, b, pl.ds(0, LANES)]
                c1 = c_v[u, b, pl.ds(LANES, LANES)]
                c2 = c_v[u, b, pl.ds(2 * LANES, LANES)]
                c3 = c_v[u, b, pl.ds(3 * LANES, LANES)]
                for l in range(L):
                    r = b * L + l
                    j = l * BC + b
                    a = (rows_v[u, j, pl.ds(0, LANES)] * c0
                         + rows_v[u, j, pl.ds(LANES, LANES)] * c1
                         + rows_v[u, j, pl.ds(2 * LANES, LANES)] * c2
                         + rows_v[u, j, pl.ds(3 * LANES, LANES)] * c3)
                    col = lax.rem(r, LANES)
                    plsc.store_scatter(tmp_v, [scat_base + col], a)

                    @pl.when(col == LANES - 1)
                    def _flush():
                        w0 = r - (LANES - 1)
                        s = tmp_v[pl.ds(0, LANES)]
                        for kk in range(1, LANES):
                            s = s + tmp_v[pl.ds(kk * LANES, LANES)]
                        rvec = w0 + iota
                        bvec = rvec // L
                        lvec = rvec - bvec * L
                        m = ((plsc.load_gather(
                                  idx_v.at[u], [lvec * BC + bvec]) != 0)
                             & (plsc.load_gather(
                                  cidx_v.at[u], [bvec]) != 0))
                        out_v[pl.ds(w0, LANES)] = jnp.where(m, s, 0.0)

                return carry

            lax.fori_loop(0, BC, b_body, 0)

        stage_idx(0, 0)
        wait_idx(0)
        fire_gathers(0, 0)

        def pair_body(h, carry):
            for u in (0, 1):
                t = h * 2 + u
                tn = lax.rem(t + 1, NCHUNK)
                r0 = (wid * BPW + t * BC) * L
                wait_pos(u)
                stage_idx(tn, 1 - u)
                table(u, p_v, pidx_v, ps_v)
                wait_idx(1 - u)
                fire_gathers(tn, 1 - u)
                wait_neg(u)
                table(u, n_v, nidx_v, ns_v)
                pltpu.sync_copy(ps_v, pdots_h.at[pl.ds(r0, RPC)])
                pltpu.sync_copy(ns_v, ndots_h.at[pl.ds(r0, RPC)])
            return carry

        lax.fori_loop(0, NCHUNK // 2, pair_body, 0)
        wait_pos(0)
        wait_neg(0)

    return k(c_all, pos_t, neg_t, center, w_context)


_ROWS, _COLS = 640, 512
_BLK = 64


def _tc_loss(pdots, ndots):
    def body(p_ref, n_ref, o_ref):
        i = pl.program_id(0)

        @pl.when(i == 0)
        def _init():
            o_ref[0, 0] = 0.0

        def ls(x):
            return jnp.minimum(x, 0.0) - jnp.log(1.0 + jnp.exp(-jnp.abs(x)))

        o_ref[0, 0] += jnp.sum(ls(p_ref[...])) + jnp.sum(ls(-n_ref[...]))

        @pl.when(i == pl.num_programs(0) - 1)
        def _fin():
            o_ref[0, 0] = o_ref[0, 0] * (-1.0 / B)

    out = pl.pallas_call(
        body,
        grid=(_ROWS // _BLK,),
        in_specs=[pl.BlockSpec((_BLK, _COLS), lambda i: (i, 0)),
                  pl.BlockSpec((_BLK, _COLS), lambda i: (i, 0))],
        out_specs=pl.BlockSpec(memory_space=pltpu.SMEM),
        out_shape=jax.ShapeDtypeStruct((1, 1), jnp.float32),
    )(pdots.reshape(_ROWS, _COLS), ndots.reshape(_ROWS, _COLS))
    return out[0, 0]


def kernel(center, pos, neg, W_center, W_context):
    c_all = _sc_center(center, W_center)
    pdots, ndots = _sc_dots(c_all, pos.T, neg.T, center, W_context)
    return _tc_loss(pdots, ndots)


# speedup vs baseline: 4.9753x; 1.1834x over previous
import functools

import jax
import jax.numpy as jnp
from jax import lax
from jax.experimental import pallas as pl
from jax.experimental.pallas import tpu as pltpu
from jax.experimental.pallas import tpu_sc as plsc

V_DIM = 1000000
D = 64
B = 16384
L = 20
LANES = 16
NC, NS = 2, 16
NW = NC * NS
BPW = B // NW
BC = 16
NCHUNK = BPW // BC
RPC = BC * L
NG = RPC // 128
_SC_PARAMS = pltpu.CompilerParams(use_tc_tiling_on_sc=False,
                                  needs_layout_passes=False)

GRP = 80
NGRP = RPC // GRP


def _sc_center(center, w_center):
    mesh = plsc.VectorSubcoreMesh(
        core_axis_name="c", subcore_axis_name="s",
        num_cores=NC, num_subcores=NS)
    GA = 16

    @functools.partial(
        pl.kernel, mesh=mesh,
        out_type=jax.ShapeDtypeStruct((B, D), jnp.float32),
        scratch_types=[
            pltpu.VMEM((BPW,), jnp.int32),
            pltpu.VMEM((GA, 8, D), jnp.float32),
            pltpu.VMEM((BPW, D), jnp.float32),
            pltpu.SemaphoreType.DMA,
        ],
        compiler_params=pltpu.CompilerParams(use_tc_tiling_on_sc=True,
                                             needs_layout_passes=False))
    def kc(center_h, wcen_h, call_h, cidx_v, tiles_v, cbuf_v, sem):
        wid = lax.axis_index("s") * NC + lax.axis_index("c")
        b0 = wid * BPW
        pltpu.sync_copy(center_h.at[pl.ds(b0, BPW)], cidx_v)

        def group(g, carry):
            cvec = cidx_v[pl.ds(g * GA, GA)]
            cps = []
            for kk in range(GA):
                tq = lax.div(cvec[kk], 8)
                cps.append(pltpu.async_copy(
                    wcen_h.at[pl.ds(tq * 8, 8), :], tiles_v.at[kk], sem))
            for cp in cps:
                cp.wait()
            for kk in range(GA):
                srow = lax.rem(cvec[kk], 8)
                j = g * GA + kk
                for k4 in range(4):
                    cbuf_v[j, pl.ds(k4 * LANES, LANES)] = (
                        tiles_v[kk, srow, pl.ds(k4 * LANES, LANES)])
            return carry

        lax.fori_loop(0, BPW // GA, group, 0)
        pltpu.sync_copy(cbuf_v, call_h.at[pl.ds(b0, BPW), :])

    return kc(center, w_center)


def _sc_dots(c_all, pos_t, neg_t, center, w_context):
    mesh = plsc.VectorSubcoreMesh(
        core_axis_name="c", subcore_axis_name="s",
        num_cores=NC, num_subcores=NS)
    out_t = (jax.ShapeDtypeStruct((B * L,), jnp.float32),
             jax.ShapeDtypeStruct((B * L,), jnp.float32))
    scratch = [
        pltpu.VMEM((2, BC), jnp.int32),
        pltpu.VMEM((2, RPC), jnp.int32),
        pltpu.VMEM((2, RPC), jnp.int32),
        pltpu.VMEM((2, BC, D), jnp.float32),
        pltpu.VMEM((2, RPC, D), jnp.float32),
        pltpu.VMEM((2, RPC, D), jnp.float32),
        pltpu.VMEM((RPC,), jnp.float32),
        pltpu.VMEM((RPC,), jnp.float32),
        pltpu.VMEM((LANES * LANES,), jnp.float32),
        pltpu.SemaphoreType.DMA,
        pltpu.SemaphoreType.DMA,
        pltpu.SemaphoreType.DMA,
        pltpu.SemaphoreType.DMA,
        pltpu.SemaphoreType.DMA,
        pltpu.SemaphoreType.DMA,
    ]

    @functools.partial(pl.kernel, out_type=out_t, mesh=mesh,
                       scratch_types=scratch,
                       compiler_params=_SC_PARAMS)
    def k(call_h, pos_h, neg_h, center_h, wctx_h, pdots_h, ndots_h,
          cidx_v, pidx_v, nidx_v, c_v, p_v, n_v, ps_v, ns_v, tmp_v,
          semp0, semp1, semn0, semn1, semi0, semi1):
        wid = lax.axis_index("s") * NC + lax.axis_index("c")
        iota = lax.iota(jnp.int32, LANES)
        scat_base = iota * LANES
        semp = [semp0, semp1]
        semn = [semn0, semn1]
        semi = [semi0, semi1]

        def stage_idx(t, u):
            b0 = wid * BPW + t * BC
            pltpu.async_copy(center_h.at[pl.ds(b0, BC)],
                             cidx_v.at[u], semi[u])
            for l in range(L):
                pltpu.async_copy(pos_h.at[l, pl.ds(b0, BC)],
                                 pidx_v.at[u, pl.ds(l * BC, BC)], semi[u])
                pltpu.async_copy(neg_h.at[l, pl.ds(b0, BC)],
                                 nidx_v.at[u, pl.ds(l * BC, BC)], semi[u])

        def wait_idx(u):
            pltpu.make_async_copy(center_h.at[pl.ds(0, BC)],
                                  cidx_v.at[u], semi[u]).wait()
            for l in range(L):
                pltpu.make_async_copy(
                    pos_h.at[0, pl.ds(0, BC)],
                    pidx_v.at[u, pl.ds(l * BC, BC)], semi[u]).wait()
                pltpu.make_async_copy(
                    neg_h.at[0, pl.ds(0, BC)],
                    nidx_v.at[u, pl.ds(l * BC, BC)], semi[u]).wait()

        def fire_gathers(t, u):
            b0 = wid * BPW + t * BC
            pltpu.async_copy(call_h.at[pl.ds(b0, BC), :],
                             c_v.at[u], semp[u])
            for g in range(NGRP):
                pltpu.async_copy(
                    wctx_h.at[pidx_v.at[u, pl.ds(g * GRP, GRP)]],
                    p_v.at[u, pl.ds(g * GRP, GRP), :], semp[u])
                pltpu.async_copy(
                    wctx_h.at[nidx_v.at[u, pl.ds(g * GRP, GRP)]],
                    n_v.at[u, pl.ds(g * GRP, GRP), :], semn[u])

        def wait_pos(u):
            pltpu.make_async_copy(call_h.at[pl.ds(0, BC), :],
                                  c_v.at[u], semp[u]).wait()
            for g in range(NGRP):
                pltpu.make_async_copy(
                    wctx_h.at[pl.ds(0, GRP), :],
                    p_v.at[u, pl.ds(g * GRP, GRP), :], semp[u]).wait()

        def wait_neg(u):
            for g in range(NGRP):
                pltpu.make_async_copy(
                    wctx_h.at[pl.ds(0, GRP), :],
                    n_v.at[u, pl.ds(g * GRP, GRP), :], semn[u]).wait()

        def table(u, rows_v, idx_v, out_v):
            def b_body(b, carry):
                c0 = c_v[u, b, pl.ds(0, LANES)]
                c1 = c_v[u, b, pl.ds(LANES, LANES)]
                c2 = c_v[u, b, pl.ds(2 * LANES, LANES)]
                c3 = c_v[u, b, pl.ds(3 * LANES, LANES)]
                for l in range(L):
                    r = b * L + l
                    j = l * BC + b
                    a = (rows_v[u, j, pl.ds(0, LANES)] * c0
                         + rows_v[u, j, pl.ds(LANES, LANES)] * c1
                         + rows_v[u, j, pl.ds(2 * LANES, LANES)] * c2
                         + rows_v[u, j, pl.ds(3 * LANES, LANES)] * c3)
                    col = lax.rem(r, LANES)
                    plsc.store_scatter(tmp_v, [scat_base + col], a)

                    @pl.when(col == LANES - 1)
                    def _flush():
                        w0 = r - (LANES - 1)
                        s = tmp_v[pl.ds(0, LANES)]
                        for kk in range(1, LANES):
                            s = s + tmp_v[pl.ds(kk * LANES, LANES)]
                        rvec = w0 + iota
                        bvec = rvec // L
                        lvec = rvec - bvec * L
                        m = ((plsc.load_gather(
                                  idx_v.at[u], [lvec * BC + bvec]) != 0)
                             & (plsc.load_gather(
                                  cidx_v.at[u], [bvec]) != 0))
                        out_v[pl.ds(w0, LANES)] = jnp.where(m, s, 0.0)

                return carry

            lax.fori_loop(0, BC, b_body, 0)

        stage_idx(0, 0)
        wait_idx(0)
        fire_gathers(0, 0)

        def pair_body(h, carry):
            for u in (0, 1):
                t = h * 2 + u
                tn = lax.rem(t + 1, NCHUNK)
                r0 = (wid * BPW + t * BC) * L
                wait_pos(u)
                stage_idx(tn, 1 - u)
                table(u, p_v, pidx_v, ps_v)
                wait_idx(1 - u)
                fire_gathers(tn, 1 - u)
                wait_neg(u)
                table(u, n_v, nidx_v, ns_v)
                pltpu.sync_copy(ps_v, pdots_h.at[pl.ds(r0, RPC)])
                pltpu.sync_copy(ns_v, ndots_h.at[pl.ds(r0, RPC)])
            return carry

        lax.fori_loop(0, NCHUNK // 2, pair_body, 0)
        wait_pos(0)
        wait_neg(0)

    return k(c_all, pos_t, neg_t, center, w_context)


_ROWS, _COLS = 640, 512
_BLK = 64


def _tc_loss(pdots, ndots):
    def body(p_ref, n_ref, o_ref):
        i = pl.program_id(0)

        @pl.when(i == 0)
        def _init():
            o_ref[0, 0] = 0.0

        def ls(x):
            return jnp.minimum(x, 0.0) - jnp.log(1.0 + jnp.exp(-jnp.abs(x)))

        o_ref[0, 0] += jnp.sum(ls(p_ref[...])) + jnp.sum(ls(-n_ref[...]))

        @pl.when(i == pl.num_programs(0) - 1)
        def _fin():
            o_ref[0, 0] = o_ref[0, 0] * (-1.0 / B)

    out = pl.pallas_call(
        body,
        grid=(_ROWS // _BLK,),
        in_specs=[pl.BlockSpec((_BLK, _COLS), lambda i: (i, 0)),
                  pl.BlockSpec((_BLK, _COLS), lambda i: (i, 0))],
        out_specs=pl.BlockSpec(memory_space=pltpu.SMEM),
        out_shape=jax.ShapeDtypeStruct((1, 1), jnp.float32),
    )(pdots.reshape(_ROWS, _COLS), ndots.reshape(_ROWS, _COLS))
    return out[0, 0]


def kernel(center, pos, neg, W_center, W_context):
    c_all = _sc_center(center, W_center)
    pdots, ndots = _sc_dots(c_all, pos.T, neg.T, center, W_context)
    return _tc_loss(pdots, ndots)
